# baseline (speedup 1.0000x reference)
import math
from functools import partial

import jax
import jax.numpy as jnp
from jax.experimental import pallas as pl
from jax.experimental.pallas import tpu as pltpu

_INV_SQRT2 = 1.0 / math.sqrt(2.0)
_RP = 3
_CP = 2


def _rup(x, m):
    return (x + m - 1) // m * m


def _pads(H, W):
    Hp, Wp = H + 2 * _RP, W + 2 * _CP
    return Hp, Wp, Hp * Wp, H * Wp, _RP * Wp


def _block_body(*args, H, Wp, O1, O2, n_pieces, clean, W):
    L = H * Wp
    base = _RP * Wp
    Cmid = 2 * O1
    Cout = 2 * O2

    xs = args[0:n_pieces]
    wc_ref = args[n_pieces]
    s1_ref, b1_ref, w2_ref, s2_ref, b2_ref, sres_ref, bres_ref, mask_ref = \
        args[n_pieces + 1:n_pieces + 9]
    main_ref = args[n_pieces + 9]
    hpad = args[n_pieces + 10]
    has_xcat = n_pieces > 1 or any(clean)
    xcat = args[n_pieces + 11] if has_xcat else None

    def tap_sum(src, O, d):
        acc = None
        for kh in range(3):
            for kw in range(3):
                t = kh * 3 + kw
                off = base + (kh - 1) * d * Wp + (kw - 1) * d
                sl = src[t * O:(t + 1) * O, off:off + L]
                acc = sl if acc is None else acc + sl
        return acc

    if xcat is None:
        xin = xs[0][...]
    else:
        off = 0
        for xr, is_clean in zip(xs, clean):
            c = xr.shape[0]
            if is_clean:
                xcat[off:off + c, :] = jnp.zeros(
                    (c, xcat.shape[1]), jnp.float32)
                for h in range(H):
                    dst = base + h * Wp + _CP
                    xcat[off:off + c, dst:dst + W] = xr[:, h * W:(h + 1) * W]
            else:
                xcat[off:off + c, :] = xr[...]
            off += c
        xin = xcat[...]
    t1 = jnp.dot(wc_ref[...], xin, preferred_element_type=jnp.float32)

    m = mask_ref[...]
    s1 = s1_ref[...]
    b1 = b1_ref[...]
    h1 = jnp.maximum(tap_sum(t1, O1, 1) * s1[0:O1] + b1[0:O1], 0.0) * m
    h2 = jnp.maximum(tap_sum(t1, O1, 2) * s1[O1:Cmid] + b1[O1:Cmid], 0.0) * m
    res = t1[9 * O1:, base:base + L] * sres_ref[...] + bres_ref[...]

    hpad[:, 0:base] = jnp.zeros((Cmid, base), jnp.float32)
    hpad[:, base + L:base + L + base] = jnp.zeros((Cmid, base), jnp.float32)
    hpad[0:O1, base:base + L] = h1
    hpad[O1:Cmid, base:base + L] = h2

    t2 = jnp.dot(w2_ref[...], hpad[...], preferred_element_type=jnp.float32)
    s2 = s2_ref[...]
    b2 = b2_ref[...]
    y1 = (tap_sum(t2, O2, 1) * s2[0:O2] + b2[0:O2] + res[0:O2]) * _INV_SQRT2
    y2 = (tap_sum(t2, O2, 2) * s2[O2:Cout] + b2[O2:Cout]
          + res[O2:Cout]) * _INV_SQRT2

    main_ref[:, 0:base] = jnp.zeros((Cout, base), jnp.float32)
    main_ref[:, base + L:base + L + base] = jnp.zeros((Cout, base), jnp.float32)
    main_ref[0:O2, base:base + L] = jnp.maximum(y1, 0.0) * m
    main_ref[O2:Cout, base:base + L] = jnp.maximum(y2, 0.0) * m


def _block(pieces, p, H, W):
    clean = [isinstance(pc, tuple) for pc in pieces]
    pieces = [pc[1] if isinstance(pc, tuple) else pc for pc in pieces]
    N = pieces[0].shape[0]
    Hp, Wp, Lp, L, _ = _pads(H, W)
    O1 = p["w1"].shape[1]
    Cmid = 2 * O1
    O2 = p["w2"].shape[1]
    Cout = 2 * O2
    Cs = [pc.shape[1] for pc in pieces]
    Cin = sum(Cs)

    w1 = p["w1"].reshape(9 * O1, Cin)
    wc = jnp.concatenate([w1, p["res_w"]], axis=0)

    col = jnp.arange(L, dtype=jnp.int32) % Wp
    mask = ((col >= _CP) & (col < _CP + W)).astype(jnp.float32).reshape(1, L)

    batch_spec = lambda shp: pl.BlockSpec(shp, lambda n: (n, 0, 0))
    const_spec = lambda shp: pl.BlockSpec(shp, lambda n: (0, 0))

    operands = list(pieces) + [wc] + [
        p["bn1_scale"].reshape(Cmid, 1), p["bn1_shift"].reshape(Cmid, 1),
        p["w2"].reshape(9 * O2, Cmid),
        p["bn2_scale"].reshape(Cout, 1), p["bn2_shift"].reshape(Cout, 1),
        p["res_scale"].reshape(Cout, 1), p["res_shift"].reshape(Cout, 1),
        mask]
    in_specs = ([batch_spec((None, c, H * W if cl else Lp))
                 for c, cl in zip(Cs, clean)]
                + [const_spec((9 * O1 + Cout, Cin))]
                + [const_spec((Cmid, 1)), const_spec((Cmid, 1)),
                   const_spec((9 * O2, Cmid)),
                   const_spec((Cout, 1)), const_spec((Cout, 1)),
                   const_spec((Cout, 1)), const_spec((Cout, 1)),
                   const_spec((1, L))])

    scratch = [pltpu.VMEM((Cmid, Lp), jnp.float32)]
    if len(pieces) > 1 or any(clean):
        scratch.append(pltpu.VMEM((Cin, Lp), jnp.float32))

    return pl.pallas_call(
        partial(_block_body, H=H, Wp=Wp, O1=O1, O2=O2, n_pieces=len(pieces),
                clean=tuple(clean), W=W),
        out_shape=jax.ShapeDtypeStruct((N, Cout, Lp), jnp.float32),
        grid=(N,),
        in_specs=in_specs,
        out_specs=batch_spec((None, Cout, Lp)),
        scratch_shapes=scratch,
        compiler_params=pltpu.CompilerParams(
            dimension_semantics=("parallel",)),
    )(*operands)


def _repad(img):
    N, C, H, W = img.shape
    _, _, Lp, _, _ = _pads(H, W)
    return jnp.pad(img, ((0, 0), (0, 0), (_RP, _RP),
                         (_CP, _CP))).reshape(N, C, Lp)


def _interior(t, H, W):
    N, C, _ = t.shape
    Hp, Wp, _, _, _ = _pads(H, W)
    return t.reshape(N, C, Hp, Wp)[:, :, _RP:_RP + H, _CP:_CP + W]


def _pool(t, H, W):
    img = _interior(t, H, W)
    N, C, _, _ = img.shape
    r = jax.image.resize(img, (N, C, H // 2, W // 2), method="bilinear",
                         antialias=False)
    return ("clean", r.reshape(N, C, H * W // 4))


def _up(t, H, W):
    img = _interior(t, H, W)
    N, C, _, _ = img.shape
    r = jax.image.resize(img, (N, C, 2 * H, 2 * W), method="bilinear",
                         antialias=False)
    return ("clean", r.reshape(N, C, 4 * H * W))


def _head_body(*args, base, L, n_pieces):
    xs = args[0:n_pieces]
    ws = args[n_pieces:2 * n_pieces]
    b_ref = args[2 * n_pieces]
    o_ref = args[2 * n_pieces + 1]
    acc = None
    for xr, wr in zip(xs, ws):
        d = jnp.dot(wr[...], xr[:, base:base + L],
                    preferred_element_type=jnp.float32)
        acc = d if acc is None else acc + d
    o_ref[...] = acc + b_ref[...]


def _head(pieces, w, b, H, W):
    N = pieces[0].shape[0]
    _, Wp, Lp, L, base = _pads(H, W)
    O = w.shape[0]
    Cs = [pc.shape[1] for pc in pieces]
    ws, off = [], 0
    for c in Cs:
        ws.append(w[:, off:off + c])
        off += c
    out = pl.pallas_call(
        partial(_head_body, base=base, L=L, n_pieces=len(pieces)),
        out_shape=jax.ShapeDtypeStruct((N, O, L), jnp.float32),
        grid=(N,),
        in_specs=([pl.BlockSpec((None, c, Lp), lambda n: (n, 0, 0))
                   for c in Cs]
                  + [pl.BlockSpec((O, c), lambda n: (0, 0)) for c in Cs]
                  + [pl.BlockSpec((O, 1), lambda n: (0, 0))]),
        out_specs=pl.BlockSpec((None, O, L), lambda n: (n, 0, 0)),
        compiler_params=pltpu.CompilerParams(
            dimension_semantics=("parallel",)),
    )(*(list(pieces) + ws + [b.reshape(O, 1)]))
    return out.reshape(N, O, H, Wp)[:, :, :, _CP:_CP + W]


_BLOCKS = ["conv0_0", "conv1_0", "conv2_0", "conv3_0", "conv4_0",
           "conv0_1", "conv1_1", "conv2_1", "conv3_1",
           "conv0_2", "conv1_2", "conv2_2",
           "conv0_3", "conv1_3", "conv0_4"]
_KEYS = ["res_w", "res_scale", "res_shift",
         "w1", "bn1_scale", "bn1_shift",
         "w2", "bn2_scale", "bn2_shift"]


def kernel(x, *flat):
    p = {}
    i = 0
    for name in _BLOCKS:
        p[name] = {k: flat[i + j] for j, k in enumerate(_KEYS)}
        i += len(_KEYS)
    final_w, final_b = flat[i], flat[i + 1]

    N, C0, H, W = x.shape
    xp = ("clean", x.reshape(N, C0, H * W))
    p00 = p["conv0_0"]

    H1, W1 = H // 2, W // 2
    H2, W2 = H // 4, W // 4
    H3, W3 = H // 8, W // 8
    H4, W4 = H // 16, W // 16

    x0_0 = _block([xp], p00, H, W)
    x1_0 = _block([_pool(x0_0, H, W)], p["conv1_0"], H1, W1)
    x0_1 = _block([x0_0, _up(x1_0, H1, W1)], p["conv0_1"], H, W)
    x2_0 = _block([_pool(x1_0, H1, W1)], p["conv2_0"], H2, W2)
    x1_1 = _block([x1_0, _up(x2_0, H2, W2)], p["conv1_1"], H1, W1)
    x0_2 = _block([x0_0, x0_1, _up(x1_1, H1, W1)], p["conv0_2"], H, W)
    x3_0 = _block([_pool(x2_0, H2, W2)], p["conv3_0"], H3, W3)
    x2_1 = _block([x2_0, _up(x3_0, H3, W3)], p["conv2_1"], H2, W2)
    x1_2 = _block([x1_0, x1_1, _up(x2_1, H2, W2)], p["conv1_2"], H1, W1)
    x0_3 = _block([x0_0, x0_1, x0_2, _up(x1_2, H1, W1)], p["conv0_3"], H, W)
    x4_0 = _block([_pool(x3_0, H3, W3)], p["conv4_0"], H4, W4)
    x3_1 = _block([x3_0, _up(x4_0, H4, W4)], p["conv3_1"], H3, W3)
    x2_2 = _block([x2_0, x2_1, _up(x3_1, H3, W3)], p["conv2_2"], H2, W2)
    x1_3 = _block([x1_0, x1_1, x1_2, _up(x2_2, H2, W2)], p["conv1_3"], H1, W1)
    x0_4 = _block([x0_0, x0_1, x0_2, x0_3, _up(x1_3, H1, W1)],
                  p["conv0_4"], H, W)
    return _head([x0_1, x0_2, x0_3, x0_4], final_w, final_b, H, W)


# speedup vs baseline: 1.0067x; 1.0067x over previous
import math
from functools import partial

import jax
import jax.numpy as jnp
from jax.experimental import pallas as pl
from jax.experimental.pallas import tpu as pltpu

_INV_SQRT2 = 1.0 / math.sqrt(2.0)
_RP = 3
_CP = 2


def _rup(x, m):
    return (x + m - 1) // m * m


def _pads(H, W):
    Hp, Wp = H + 2 * _RP, W + 2 * _CP
    return Hp, Wp, Hp * Wp, H * Wp, _RP * Wp


def _block_body(*args, H, Wp, O1, O2, n_pieces, clean, W):
    L = H * Wp
    base = _RP * Wp
    Cmid = 2 * O1
    Cout = 2 * O2

    xs = args[0:n_pieces]
    wc_ref = args[n_pieces]
    s1_ref, b1_ref, w2_ref, s2_ref, b2_ref, sres_ref, bres_ref, mask_ref = \
        args[n_pieces + 1:n_pieces + 9]
    main_ref = args[n_pieces + 9]
    hpad = args[n_pieces + 10]
    has_xcat = n_pieces > 1 or any(clean)
    xcat = args[n_pieces + 11] if has_xcat else None

    def tap_sum(src, O, d):
        acc = None
        for kh in range(3):
            for kw in range(3):
                t = kh * 3 + kw
                off = base + (kh - 1) * d * Wp + (kw - 1) * d
                sl = src[t * O:(t + 1) * O, off:off + L]
                acc = sl if acc is None else acc + sl
        return acc

    if xcat is None:
        xin = xs[0][...]
    else:
        off = 0
        for xr, is_clean in zip(xs, clean):
            c = xr.shape[0]
            if is_clean:
                xcat[off:off + c, :] = jnp.zeros(
                    (c, xcat.shape[1]), jnp.float32)
                for h in range(H):
                    dst = base + h * Wp + _CP
                    xcat[off:off + c, dst:dst + W] = xr[:, h * W:(h + 1) * W]
            else:
                xcat[off:off + c, :] = xr[...]
            off += c
        xin = xcat[...]
    t1 = jnp.dot(wc_ref[...], xin, preferred_element_type=jnp.float32)

    m = mask_ref[...]
    s1 = s1_ref[...]
    b1 = b1_ref[...]
    h1 = jnp.maximum(tap_sum(t1, O1, 1) * s1[0:O1] + b1[0:O1], 0.0) * m
    h2 = jnp.maximum(tap_sum(t1, O1, 2) * s1[O1:Cmid] + b1[O1:Cmid], 0.0) * m
    res = t1[9 * O1:, base:base + L] * sres_ref[...] + bres_ref[...]

    hpad[:, 0:base] = jnp.zeros((Cmid, base), jnp.float32)
    hpad[:, base + L:base + L + base] = jnp.zeros((Cmid, base), jnp.float32)
    hpad[0:O1, base:base + L] = h1
    hpad[O1:Cmid, base:base + L] = h2

    t2 = jnp.dot(w2_ref[...], hpad[...], preferred_element_type=jnp.float32)
    s2 = s2_ref[...]
    b2 = b2_ref[...]
    y1 = (tap_sum(t2, O2, 1) * s2[0:O2] + b2[0:O2] + res[0:O2]) * _INV_SQRT2
    y2 = (tap_sum(t2, O2, 2) * s2[O2:Cout] + b2[O2:Cout]
          + res[O2:Cout]) * _INV_SQRT2

    main_ref[:, 0:base] = jnp.zeros((Cout, base), jnp.float32)
    main_ref[:, base + L:base + L + base] = jnp.zeros((Cout, base), jnp.float32)
    main_ref[0:O2, base:base + L] = jnp.maximum(y1, 0.0) * m
    main_ref[O2:Cout, base:base + L] = jnp.maximum(y2, 0.0) * m


def _block(pieces, p, H, W):
    clean = [isinstance(pc, tuple) for pc in pieces]
    pieces = [pc[1] if isinstance(pc, tuple) else pc for pc in pieces]
    N = pieces[0].shape[0]
    Hp, Wp, Lp, L, _ = _pads(H, W)
    O1 = p["w1"].shape[1]
    Cmid = 2 * O1
    O2 = p["w2"].shape[1]
    Cout = 2 * O2
    Cs = [pc.shape[1] for pc in pieces]
    Cin = sum(Cs)

    w1 = p["w1"].reshape(9 * O1, Cin)
    wc = jnp.concatenate([w1, p["res_w"]], axis=0)

    col = jnp.arange(L, dtype=jnp.int32) % Wp
    mask = ((col >= _CP) & (col < _CP + W)).astype(jnp.float32).reshape(1, L)

    batch_spec = lambda shp: pl.BlockSpec(shp, lambda n: (n, 0, 0))
    const_spec = lambda shp: pl.BlockSpec(shp, lambda n: (0, 0))

    operands = list(pieces) + [wc] + [
        p["bn1_scale"].reshape(Cmid, 1), p["bn1_shift"].reshape(Cmid, 1),
        p["w2"].reshape(9 * O2, Cmid),
        p["bn2_scale"].reshape(Cout, 1), p["bn2_shift"].reshape(Cout, 1),
        p["res_scale"].reshape(Cout, 1), p["res_shift"].reshape(Cout, 1),
        mask]
    in_specs = ([batch_spec((None, c, H * W if cl else Lp))
                 for c, cl in zip(Cs, clean)]
                + [const_spec((9 * O1 + Cout, Cin))]
                + [const_spec((Cmid, 1)), const_spec((Cmid, 1)),
                   const_spec((9 * O2, Cmid)),
                   const_spec((Cout, 1)), const_spec((Cout, 1)),
                   const_spec((Cout, 1)), const_spec((Cout, 1)),
                   const_spec((1, L))])

    scratch = [pltpu.VMEM((Cmid, Lp), jnp.float32)]
    if len(pieces) > 1 or any(clean):
        scratch.append(pltpu.VMEM((Cin, Lp), jnp.float32))

    return pl.pallas_call(
        partial(_block_body, H=H, Wp=Wp, O1=O1, O2=O2, n_pieces=len(pieces),
                clean=tuple(clean), W=W),
        out_shape=jax.ShapeDtypeStruct((N, Cout, Lp), jnp.float32),
        grid=(N,),
        in_specs=in_specs,
        out_specs=batch_spec((None, Cout, Lp)),
        scratch_shapes=scratch,
        compiler_params=pltpu.CompilerParams(
            dimension_semantics=("parallel",)),
    )(*operands)


def _repad(img):
    N, C, H, W = img.shape
    _, _, Lp, _, _ = _pads(H, W)
    return jnp.pad(img, ((0, 0), (0, 0), (_RP, _RP),
                         (_CP, _CP))).reshape(N, C, Lp)


def _interior(t, H, W):
    N, C, _ = t.shape
    Hp, Wp, _, _, _ = _pads(H, W)
    return t.reshape(N, C, Hp, Wp)[:, :, _RP:_RP + H, _CP:_CP + W]


def _pool(t, H, W):
    img = _interior(t, H, W)
    N, C, _, _ = img.shape
    r = jax.image.resize(img, (N, C, H // 2, W // 2), method="bilinear",
                         antialias=False)
    return _repad(r)


def _up(t, H, W):
    img = _interior(t, H, W)
    N, C, _, _ = img.shape
    r = jax.image.resize(img, (N, C, 2 * H, 2 * W), method="bilinear",
                         antialias=False)
    return ("clean", r.reshape(N, C, 4 * H * W))


def _head_body(*args, base, L, n_pieces):
    xs = args[0:n_pieces]
    ws = args[n_pieces:2 * n_pieces]
    b_ref = args[2 * n_pieces]
    o_ref = args[2 * n_pieces + 1]
    acc = None
    for xr, wr in zip(xs, ws):
        d = jnp.dot(wr[...], xr[:, base:base + L],
                    preferred_element_type=jnp.float32)
        acc = d if acc is None else acc + d
    o_ref[...] = acc + b_ref[...]


def _head(pieces, w, b, H, W):
    N = pieces[0].shape[0]
    _, Wp, Lp, L, base = _pads(H, W)
    O = w.shape[0]
    Cs = [pc.shape[1] for pc in pieces]
    ws, off = [], 0
    for c in Cs:
        ws.append(w[:, off:off + c])
        off += c
    out = pl.pallas_call(
        partial(_head_body, base=base, L=L, n_pieces=len(pieces)),
        out_shape=jax.ShapeDtypeStruct((N, O, L), jnp.float32),
        grid=(N,),
        in_specs=([pl.BlockSpec((None, c, Lp), lambda n: (n, 0, 0))
                   for c in Cs]
                  + [pl.BlockSpec((O, c), lambda n: (0, 0)) for c in Cs]
                  + [pl.BlockSpec((O, 1), lambda n: (0, 0))]),
        out_specs=pl.BlockSpec((None, O, L), lambda n: (n, 0, 0)),
        compiler_params=pltpu.CompilerParams(
            dimension_semantics=("parallel",)),
    )(*(list(pieces) + ws + [b.reshape(O, 1)]))
    return out.reshape(N, O, H, Wp)[:, :, :, _CP:_CP + W]


_BLOCKS = ["conv0_0", "conv1_0", "conv2_0", "conv3_0", "conv4_0",
           "conv0_1", "conv1_1", "conv2_1", "conv3_1",
           "conv0_2", "conv1_2", "conv2_2",
           "conv0_3", "conv1_3", "conv0_4"]
_KEYS = ["res_w", "res_scale", "res_shift",
         "w1", "bn1_scale", "bn1_shift",
         "w2", "bn2_scale", "bn2_shift"]


def kernel(x, *flat):
    p = {}
    i = 0
    for name in _BLOCKS:
        p[name] = {k: flat[i + j] for j, k in enumerate(_KEYS)}
        i += len(_KEYS)
    final_w, final_b = flat[i], flat[i + 1]

    N, C0, H, W = x.shape
    C0p = _rup(C0, 8)
    _, _, Lp0, _, _ = _pads(H, W)
    xp = jnp.pad(x, ((0, 0), (0, C0p - C0),
                     (_RP, _RP), (_CP, _CP))).reshape(N, C0p, Lp0)
    p00 = dict(p["conv0_0"])
    p00["w1"] = jnp.pad(p00["w1"], ((0, 0), (0, 0), (0, C0p - C0)))
    p00["res_w"] = jnp.pad(p00["res_w"], ((0, 0), (0, C0p - C0)))

    H1, W1 = H // 2, W // 2
    H2, W2 = H // 4, W // 4
    H3, W3 = H // 8, W // 8
    H4, W4 = H // 16, W // 16

    x0_0 = _block([xp], p00, H, W)
    x1_0 = _block([_pool(x0_0, H, W)], p["conv1_0"], H1, W1)
    x0_1 = _block([x0_0, _up(x1_0, H1, W1)], p["conv0_1"], H, W)
    x2_0 = _block([_pool(x1_0, H1, W1)], p["conv2_0"], H2, W2)
    x1_1 = _block([x1_0, _up(x2_0, H2, W2)], p["conv1_1"], H1, W1)
    x0_2 = _block([x0_0, x0_1, _up(x1_1, H1, W1)], p["conv0_2"], H, W)
    x3_0 = _block([_pool(x2_0, H2, W2)], p["conv3_0"], H3, W3)
    x2_1 = _block([x2_0, _up(x3_0, H3, W3)], p["conv2_1"], H2, W2)
    x1_2 = _block([x1_0, x1_1, _up(x2_1, H2, W2)], p["conv1_2"], H1, W1)
    x0_3 = _block([x0_0, x0_1, x0_2, _up(x1_2, H1, W1)], p["conv0_3"], H, W)
    x4_0 = _block([_pool(x3_0, H3, W3)], p["conv4_0"], H4, W4)
    x3_1 = _block([x3_0, _up(x4_0, H4, W4)], p["conv3_1"], H3, W3)
    x2_2 = _block([x2_0, x2_1, _up(x3_1, H3, W3)], p["conv2_2"], H2, W2)
    x1_3 = _block([x1_0, x1_1, x1_2, _up(x2_2, H2, W2)], p["conv1_3"], H1, W1)
    x0_4 = _block([x0_0, x0_1, x0_2, x0_3, _up(x1_3, H1, W1)],
                  p["conv0_4"], H, W)
    return _head([x0_1, x0_2, x0_3, x0_4], final_w, final_b, H, W)
